# Initial kernel scaffold; baseline (speedup 1.0000x reference)
import jax
import jax.numpy as jnp
from jax.experimental import pallas as pl


def kernel(x, emb, w_lin, b_lin, W1, b1, W2, b2, W3, b3):
    raise NotImplementedError("write your pallas kernel here")


# speedup vs baseline: 30.3520x; 30.3520x over previous
import functools

import numpy as np
import jax
import jax.numpy as jnp
from jax import lax
from jax.experimental import pallas as pl
from jax.experimental.pallas import tpu as pltpu
from jax.experimental.pallas import tpu_sc as plsc

_FIELD_DIMS = [1000] * 26
_F = 26
_FEAT = sum(_FIELD_DIMS)
_D = 16
_PAIRS = _F * (_F - 1) // 2
_INTER = _PAIRS * _D
_INTER_PAD = 5248
_B = 4096
_ROW = _F * _D + 16
_WCOL = _F * _D
_OFFS = np.asarray([0, *np.cumsum(_FIELD_DIMS)[:-1]], dtype=np.int32)

_NW = 32
_BPW = _B // _NW


def _sc_make():
    mesh = plsc.VectorSubcoreMesh(core_axis_name="c", subcore_axis_name="s")

    @functools.partial(
        pl.kernel,
        mesh=mesh,
        compiler_params=pltpu.CompilerParams(use_tc_tiling_on_sc=False),
        out_type=[
            jax.ShapeDtypeStruct((_B, _INTER_PAD), jnp.float32),
            jax.ShapeDtypeStruct((_B, 16), jnp.float32),
        ],
        scratch_types=[
            pltpu.VMEM((_BPW, _F), jnp.int32),
            pltpu.VMEM((2, _F, _ROW), jnp.float32),
            pltpu.VMEM((2, _INTER_PAD), jnp.float32),
            pltpu.VMEM((_BPW, 16), jnp.float32),
            pltpu.SemaphoreType.DMA,
            pltpu.SemaphoreType.DMA,
            pltpu.SemaphoreType.DMA,
            pltpu.SemaphoreType.DMA,
        ],
    )
    def sc_interactions(T_hbm, idx_hbm, h_hbm, fo_hbm,
                        idx_v, rows_v, h_v, fo_v,
                        gsem0, gsem1, wsem0, wsem1):
        wid = lax.axis_index("s") * 2 + lax.axis_index("c")
        base = wid * _BPW
        pltpu.sync_copy(idx_hbm.at[pl.ds(base, _BPW)], idx_v)

        zero16 = jnp.zeros((16,), jnp.float32)
        for slot in (0, 1):
            for j in range(3):
                h_v[slot, pl.ds(_INTER + j * 16, 16)] = zero16

        pltpu.async_copy(T_hbm.at[idx_v.at[0]], rows_v.at[0], gsem0)
        pltpu.async_copy(T_hbm.at[idx_v.at[1]], rows_v.at[1], gsem1)

        def half_step(b, slot, gsem, wsem):
            pltpu.make_async_copy(
                T_hbm.at[idx_v.at[b]], rows_v.at[slot], gsem).wait()
            @pl.when(b >= 2)
            def _():
                pltpu.make_async_copy(
                    h_v.at[slot], h_hbm.at[base + b - 2], wsem).wait()

            def f_loop(f, p):
                def g_loop(g, p):
                    va = rows_v[slot, g, pl.ds(pl.multiple_of(f * _D, 16), 16)]
                    vb = rows_v[slot, f, pl.ds(pl.multiple_of(g * _D, 16), 16)]
                    h_v[slot, pl.ds(pl.multiple_of(p, 16), 16)] = va * vb
                    return p + 16
                return lax.fori_loop(f + 1, _F, g_loop, p)
            lax.fori_loop(0, _F - 1, f_loop, 0)

            def w_loop(f, acc):
                return acc + rows_v[slot, f, pl.ds(_WCOL, 16)]
            fo_v[b, :] = lax.fori_loop(
                0, _F, w_loop, jnp.zeros((16,), jnp.float32))

            @pl.when(b + 2 < _BPW)
            def _():
                pltpu.async_copy(
                    T_hbm.at[idx_v.at[b + 2]], rows_v.at[slot], gsem)
            pltpu.async_copy(h_v.at[slot], h_hbm.at[base + b], wsem)

        def iter_body(i, _):
            half_step(2 * i, 0, gsem0, wsem0)
            half_step(2 * i + 1, 1, gsem1, wsem1)
            return 0
        lax.fori_loop(0, _BPW // 2, iter_body, 0)

        pltpu.make_async_copy(
            h_v.at[0], h_hbm.at[base + _BPW - 2], wsem0).wait()
        pltpu.make_async_copy(
            h_v.at[1], h_hbm.at[base + _BPW - 1], wsem1).wait()

        pltpu.sync_copy(fo_v, fo_hbm.at[pl.ds(base, _BPW)])

    return sc_interactions


_sc_interactions = _sc_make()

_BT = 512


def _mlp_body(h_ref, fo_ref, W1_ref, b1_ref, W2_ref, b2_ref, W3_ref, b3_ref,
              out_ref):
    a1 = jnp.dot(h_ref[...], W1_ref[...], preferred_element_type=jnp.float32)
    a1 = jnp.maximum(a1 + b1_ref[...], 0.0)
    a2 = jnp.dot(a1, W2_ref[...], preferred_element_type=jnp.float32)
    a2 = jnp.maximum(a2 + b2_ref[...], 0.0)
    a3 = jnp.dot(a2, W3_ref[...], preferred_element_type=jnp.float32)
    fo = jnp.sum(fo_ref[...], axis=1, keepdims=True)
    out_ref[...] = a3 + fo + b3_ref[...]


_mlp_call = pl.pallas_call(
    _mlp_body,
    grid=(_B // _BT,),
    in_specs=[
        pl.BlockSpec((_BT, _INTER_PAD), lambda i: (i, 0)),
        pl.BlockSpec((_BT, 16), lambda i: (i, 0)),
        pl.BlockSpec((_INTER_PAD, 64), lambda i: (0, 0)),
        pl.BlockSpec((1, 64), lambda i: (0, 0)),
        pl.BlockSpec((64, 32), lambda i: (0, 0)),
        pl.BlockSpec((1, 32), lambda i: (0, 0)),
        pl.BlockSpec((32, 1), lambda i: (0, 0)),
        pl.BlockSpec((1, 1), lambda i: (0, 0)),
    ],
    out_specs=pl.BlockSpec((_BT, 1), lambda i: (i, 0)),
    out_shape=jax.ShapeDtypeStruct((_B, 1), jnp.float32),
)


def kernel(x, emb, w_lin, b_lin, W1, b1, W2, b2, W3, b3):
    x_off = x + jnp.asarray(_OFFS)[None, :]
    T = jnp.concatenate(
        [
            jnp.transpose(emb, (1, 0, 2)).reshape(_FEAT, _F * _D),
            w_lin.reshape(_FEAT, 1),
            jnp.zeros((_FEAT, _ROW - _WCOL - 1), jnp.float32),
        ],
        axis=1,
    )
    h, fo = _sc_interactions(T, x_off)
    W1p = jnp.concatenate([W1, jnp.zeros((_INTER_PAD - _INTER, 64), jnp.float32)], axis=0)
    out = _mlp_call(h, fo, W1p, b1.reshape(1, 64), W2, b2.reshape(1, 32),
                    W3, (b3 + b_lin).reshape(1, 1))
    return out[:, 0]
